# Initial kernel scaffold; baseline (speedup 1.0000x reference)
import jax
import jax.numpy as jnp
from jax.experimental import pallas as pl


def kernel(inputs, pos_table):
    raise NotImplementedError("write your pallas kernel here")


# speedup vs baseline: 1.6665x; 1.6665x over previous
import math

import jax
import jax.numpy as jnp
from jax.experimental import pallas as pl


_SCALE = math.sqrt(4096.0)
_S_BLK = 256


def _add_pos_kernel(x_ref, pos_ref, o_ref):
    o_ref[...] = x_ref[...] * _SCALE + pos_ref[...][None, :, :]


@jax.jit
def kernel(inputs, pos_table):
    b, s, d = inputs.shape
    s_blocks = s // _S_BLK
    return pl.pallas_call(
        _add_pos_kernel,
        grid=(s_blocks, b),
        in_specs=[
            pl.BlockSpec((1, _S_BLK, d), lambda i, j: (j, i, 0)),
            pl.BlockSpec((_S_BLK, d), lambda i, j: (i, 0)),
        ],
        out_specs=pl.BlockSpec((1, _S_BLK, d), lambda i, j: (j, i, 0)),
        out_shape=jax.ShapeDtypeStruct((b, s, d), inputs.dtype),
    )(inputs, pos_table)


# speedup vs baseline: 1.7208x; 1.0326x over previous
import math

import jax
import jax.numpy as jnp
from jax.experimental import pallas as pl


_SCALE = math.sqrt(4096.0)
_S_BLK = 128


def _add_pos_kernel(x_ref, pos_ref, o_ref):
    o_ref[...] = x_ref[...] * _SCALE + pos_ref[...][None, :, :]


@jax.jit
def kernel(inputs, pos_table):
    b, s, d = inputs.shape
    s_blocks = s // _S_BLK
    return pl.pallas_call(
        _add_pos_kernel,
        grid=(s_blocks,),
        in_specs=[
            pl.BlockSpec((b, _S_BLK, d), lambda i: (0, i, 0)),
            pl.BlockSpec((_S_BLK, d), lambda i: (i, 0)),
        ],
        out_specs=pl.BlockSpec((b, _S_BLK, d), lambda i: (0, i, 0)),
        out_shape=jax.ShapeDtypeStruct((b, s, d), inputs.dtype),
    )(inputs, pos_table)


# speedup vs baseline: 1.7385x; 1.0103x over previous
import math

import jax
import jax.numpy as jnp
from jax.experimental import pallas as pl
from jax.experimental.pallas import tpu as pltpu


_SCALE = math.sqrt(4096.0)
_S_BLK = 256
_B_BLK = 2


def _add_pos_kernel(x_ref, pos_ref, o_ref):
    o_ref[...] = x_ref[...] * _SCALE + pos_ref[...][None, :, :]


@jax.jit
def kernel(inputs, pos_table):
    b, s, d = inputs.shape
    s_blocks = s // _S_BLK
    return pl.pallas_call(
        _add_pos_kernel,
        grid=(s_blocks, b // _B_BLK),
        in_specs=[
            pl.BlockSpec((_B_BLK, _S_BLK, d), lambda i, j: (j, i, 0)),
            pl.BlockSpec((_S_BLK, d), lambda i, j: (i, 0)),
        ],
        out_specs=pl.BlockSpec((_B_BLK, _S_BLK, d), lambda i, j: (j, i, 0)),
        out_shape=jax.ShapeDtypeStruct((b, s, d), inputs.dtype),
    )(inputs, pos_table)
